# baseline (speedup 1.0000x reference)
import functools

import jax
import jax.numpy as jnp
from jax import lax
from jax.experimental import pallas as pl
from jax.experimental.pallas import tpu as pltpu
from jax.experimental.pallas import tpu_sc as plsc

NC, NS = 2, 16
NW = NC * NS
B = 16384
D = 64
L = 16
W = 2 * D
HALF = 500000
BPW = B // NW
IDX_ROWS = B // 128 // NW
NCHUNK = 4
CH = BPW // NCHUNK


def _sc_dot_body(u_hbm, i_hbm, ut_hbm, it_hbm, out_hbm,
                 uidx, iidx, ugidx, igidx, urows, irows, wbuf, outv, sem):
    wid = lax.axis_index("s") * NC + lax.axis_index("c")
    row0 = wid * IDX_ROWS

    pltpu.sync_copy(u_hbm.at[pl.ds(row0, IDX_ROWS)], uidx)
    pltpu.sync_copy(i_hbm.at[pl.ds(row0, IDX_ROWS)], iidx)

    for q in range(IDX_ROWS):
        for k in range(128 // L):
            uv = uidx[q, pl.ds(k * L, L)]
            iv = iidx[q, pl.ds(k * L, L)]
            ugidx[q, pl.ds(k * L, L)] = uv - jnp.where(uv >= HALF, HALF, 0)
            igidx[q, pl.ds(k * L, L)] = iv - jnp.where(iv >= HALF, HALF, 0)

    for q in range(NCHUNK):
        cp_u = pltpu.async_copy(ut_hbm.at[ugidx.at[q]], urows, sem)
        cp_i = pltpu.async_copy(it_hbm.at[igidx.at[q]], irows, sem)
        cp_u.wait()
        cp_i.wait()

        def chunk_body(t, carry, q=q):
            hu = jnp.where(uidx[q, pl.ds(t * L, L)] >= HALF, D, 0)
            hi = jnp.where(iidx[q, pl.ds(t * L, L)] >= HALF, D, 0)
            for j in range(L):
                ou = hu[j]
                oi = hi[j]
                r = t * L + j
                w = urows[r, pl.ds(ou, L)] * irows[r, pl.ds(oi, L)]
                w += urows[r, pl.ds(ou + L, L)] * irows[r, pl.ds(oi + L, L)]
                w += (urows[r, pl.ds(ou + 2 * L, L)]
                      * irows[r, pl.ds(oi + 2 * L, L)])
                w += (urows[r, pl.ds(ou + 3 * L, L)]
                      * irows[r, pl.ds(oi + 3 * L, L)])
                wbuf[pl.ds(((q * CH) + r) * L, L)] = w
            return carry
        lax.fori_loop(0, CH // L, chunk_body, 0)

    def grp_body(g, carry):
        j0 = g * L
        base_ids = (j0 + lax.iota(jnp.int32, L)) * L
        acc = plsc.load_gather(wbuf, [base_ids])
        for l in range(1, L):
            acc += plsc.load_gather(wbuf, [base_ids + l])
        outv[pl.ds(j0, L)] = acc
        return carry
    lax.fori_loop(0, BPW // L, grp_body, 0)

    pltpu.sync_copy(outv, out_hbm.at[pl.ds(wid * BPW, BPW)])


def _tc_pack_body(a_ref, b_ref, o_ref):
    o_ref[:, 0:D] = a_ref[...]
    o_ref[:, D:W] = b_ref[...]


def _tc_pack(x):
    n = x.shape[0]
    blk = 4000
    steps = (n // 2) // blk
    return pl.pallas_call(
        _tc_pack_body,
        grid=(steps,),
        in_specs=[
            pl.BlockSpec((blk, D), lambda i: (i, 0)),
            pl.BlockSpec((blk, D), lambda i, s=steps: (i + s, 0)),
        ],
        out_specs=pl.BlockSpec((blk, W), lambda i: (i, 0)),
        out_shape=jax.ShapeDtypeStruct((n // 2, W), x.dtype),
    )(x, x)


def kernel(user, item, user_table, item_table):
    user2d = user.reshape(128, 128)
    item2d = item.reshape(128, 128)
    ut2 = _tc_pack(user_table)
    it2 = _tc_pack(item_table)
    mesh = plsc.VectorSubcoreMesh(core_axis_name="c", subcore_axis_name="s")
    out = pl.kernel(
        _sc_dot_body,
        out_type=jax.ShapeDtypeStruct((B,), jnp.float32),
        mesh=mesh,
        compiler_params=pltpu.CompilerParams(needs_layout_passes=False),
        scratch_types=[
            pltpu.VMEM((IDX_ROWS, 128), jnp.int32),
            pltpu.VMEM((IDX_ROWS, 128), jnp.int32),
            pltpu.VMEM((NCHUNK, CH), jnp.int32),
            pltpu.VMEM((NCHUNK, CH), jnp.int32),
            pltpu.VMEM((CH, W), jnp.float32),
            pltpu.VMEM((CH, W), jnp.float32),
            pltpu.VMEM((BPW * L,), jnp.float32),
            pltpu.VMEM((BPW,), jnp.float32),
            pltpu.SemaphoreType.DMA,
        ],
    )(user2d, item2d, ut2, it2)
    return out


# speedup vs baseline: 1.7137x; 1.7137x over previous
import functools

import jax
import jax.numpy as jnp
from jax import lax
from jax.experimental import pallas as pl
from jax.experimental.pallas import tpu as pltpu
from jax.experimental.pallas import tpu_sc as plsc

NC, NS = 2, 16
NW = NC * NS
B = 16384
D = 64
L = 16
BPW = B // NW
IDX_ROWS = B // 128 // NW


def _sc_dot_body(u_hbm, i_hbm, ut_hbm, it_hbm, out_hbm,
                 uidx, iidx, ubuf, ibuf, wbuf, outv, sem):
    wid = lax.axis_index("s") * NC + lax.axis_index("c")
    row0 = wid * IDX_ROWS

    pltpu.sync_copy(u_hbm.at[pl.ds(row0, IDX_ROWS)], uidx)
    pltpu.sync_copy(i_hbm.at[pl.ds(row0, IDX_ROWS)], iidx)

    def fire(t, carry):
        uv = uidx[t // 8, pl.ds((t % 8) * L, L)]
        iv = iidx[t // 8, pl.ds((t % 8) * L, L)]
        for j in range(L):
            p = t * (L // 2) + j // 2
            h = (j % 2) * D
            pltpu.async_copy(ut_hbm.at[uv[j]], ubuf.at[p, pl.ds(h, D)], sem)
            pltpu.async_copy(it_hbm.at[iv[j]], ibuf.at[p, pl.ds(h, D)], sem)
        return carry
    lax.fori_loop(0, BPW // L, fire, 0)

    def drain(t, carry):
        pltpu.make_async_copy(ut_hbm.at[0], ubuf.at[0, pl.ds(0, D)], sem).wait()
        pltpu.make_async_copy(it_hbm.at[0], ibuf.at[0, pl.ds(0, D)], sem).wait()
        return carry
    lax.fori_loop(0, BPW, drain, 0)

    def pair_body(p, carry):
        for h in range(2):
            o = h * D
            w = ubuf[p, pl.ds(o, L)] * ibuf[p, pl.ds(o, L)]
            w += ubuf[p, pl.ds(o + L, L)] * ibuf[p, pl.ds(o + L, L)]
            w += ubuf[p, pl.ds(o + 2 * L, L)] * ibuf[p, pl.ds(o + 2 * L, L)]
            w += ubuf[p, pl.ds(o + 3 * L, L)] * ibuf[p, pl.ds(o + 3 * L, L)]
            wbuf[pl.ds((2 * p + h) * L, L)] = w
        return carry
    lax.fori_loop(0, BPW // 2, pair_body, 0)

    def grp_body(g, carry):
        j0 = g * L
        base_ids = (j0 + lax.iota(jnp.int32, L)) * L
        acc = plsc.load_gather(wbuf, [base_ids])
        for l in range(1, L):
            acc += plsc.load_gather(wbuf, [base_ids + l])
        outv[pl.ds(j0, L)] = acc
        return carry
    lax.fori_loop(0, BPW // L, grp_body, 0)

    pltpu.sync_copy(outv, out_hbm.at[pl.ds(wid * BPW, BPW)])


def kernel(user, item, user_table, item_table):
    user2d = user.reshape(128, 128)
    item2d = item.reshape(128, 128)
    mesh = plsc.VectorSubcoreMesh(core_axis_name="c", subcore_axis_name="s")
    out = pl.kernel(
        _sc_dot_body,
        out_type=jax.ShapeDtypeStruct((B,), jnp.float32),
        mesh=mesh,
        compiler_params=pltpu.CompilerParams(needs_layout_passes=False),
        scratch_types=[
            pltpu.VMEM((IDX_ROWS, 128), jnp.int32),
            pltpu.VMEM((IDX_ROWS, 128), jnp.int32),
            pltpu.VMEM((BPW // 2, 2 * D), jnp.float32),
            pltpu.VMEM((BPW // 2, 2 * D), jnp.float32),
            pltpu.VMEM((BPW * L,), jnp.float32),
            pltpu.VMEM((BPW,), jnp.float32),
            pltpu.SemaphoreType.DMA,
        ],
    )(user2d, item2d, user_table, item_table)
    return out
